# baseline (speedup 1.0000x reference)
import jax
import jax.numpy as jnp
from jax import lax
from jax.experimental import pallas as pl
from jax.experimental.pallas import tpu as pltpu
from jax.experimental.pallas import tpu_sc as plsc

B = 4096
S = 50
D = 768
NC = 2
NS = 16
NW = NC * NS
B_PER_W = B // NW


CH = 8
CB = B_PER_W // CH


def _body(xt_ref, so_ref, out_ref, dst_idx, rows, sem_g, sem_i, sem_s):
    wid = lax.axis_index("s") * NC + lax.axis_index("c")
    base = wid * B_PER_W
    gathers = [
        pltpu.async_copy(
            xt_ref.at[0, pl.ds(base + k * CB, CB)],
            rows.at[pl.ds(k * CB, CB)],
            sem_g.at[k],
        )
        for k in range(CH)
    ]
    idx_copies = [
        pltpu.async_copy(so_ref.at[pl.ds(base + k * CB, CB)], dst_idx.at[k], sem_i)
        for k in range(CH)
    ]
    for c in idx_copies:
        c.wait()
    scatters = []
    for k in range(CH):
        gathers[k].wait()
        scatters.append(
            pltpu.async_copy(rows.at[pl.ds(k * CB, CB)], out_ref.at[dst_idx.at[k]], sem_s)
        )
    for c in scatters:
        c.wait()


def kernel(embeded_big_tensor, sorted_lengths, sort_order, sentences_per_doc):
    xt = embeded_big_tensor.transpose(1, 0, 2)
    so = sort_order.astype(jnp.int32)
    mesh = plsc.VectorSubcoreMesh(core_axis_name="c", subcore_axis_name="s")
    out = pl.kernel(
        _body,
        out_type=jax.ShapeDtypeStruct((B, D), jnp.float32),
        mesh=mesh,
        scratch_types=[
            pltpu.VMEM((CH, CB), jnp.int32),
            pltpu.VMEM((B_PER_W, D), jnp.float32),
            pltpu.SemaphoreType.DMA((CH,)),
            pltpu.SemaphoreType.DMA,
            pltpu.SemaphoreType.DMA,
        ],
        compiler_params=pltpu.CompilerParams(
            skip_device_barrier=True,
            disable_bounds_checks=True,
            disable_semaphore_checks=True,
        ),
    )(xt, so)
    return (out, sentences_per_doc)


# speedup vs baseline: 1.0063x; 1.0063x over previous
import jax
import jax.numpy as jnp
from jax import lax
from jax.experimental import pallas as pl
from jax.experimental.pallas import tpu as pltpu
from jax.experimental.pallas import tpu_sc as plsc

B = 4096
S = 50
D = 768
NC = 2
NS = 16
NW = NC * NS
B_PER_W = B // NW


CH = 2
CB = B_PER_W // CH


def _body(xt_ref, so_ref, out_ref, dst_idx, rows, sem_g, sem_i, sem_s):
    wid = lax.axis_index("s") * NC + lax.axis_index("c")
    base = wid * B_PER_W
    gathers = [
        pltpu.async_copy(
            xt_ref.at[0, pl.ds(base + k * CB, CB)],
            rows.at[pl.ds(k * CB, CB)],
            sem_g.at[k],
        )
        for k in range(CH)
    ]
    idx_copies = [
        pltpu.async_copy(so_ref.at[pl.ds(base + k * CB, CB)], dst_idx.at[k], sem_i)
        for k in range(CH)
    ]
    for c in idx_copies:
        c.wait()
    scatters = []
    for k in range(CH):
        gathers[k].wait()
        scatters.append(
            pltpu.async_copy(rows.at[pl.ds(k * CB, CB)], out_ref.at[dst_idx.at[k]], sem_s)
        )
    for c in scatters:
        c.wait()


def kernel(embeded_big_tensor, sorted_lengths, sort_order, sentences_per_doc):
    xt = embeded_big_tensor.transpose(1, 0, 2)
    so = sort_order.astype(jnp.int32)
    mesh = plsc.VectorSubcoreMesh(core_axis_name="c", subcore_axis_name="s")
    out = pl.kernel(
        _body,
        out_type=jax.ShapeDtypeStruct((B, D), jnp.float32),
        mesh=mesh,
        scratch_types=[
            pltpu.VMEM((CH, CB), jnp.int32),
            pltpu.VMEM((B_PER_W, D), jnp.float32),
            pltpu.SemaphoreType.DMA((CH,)),
            pltpu.SemaphoreType.DMA,
            pltpu.SemaphoreType.DMA,
        ],
        compiler_params=pltpu.CompilerParams(
            skip_device_barrier=True,
            disable_bounds_checks=True,
            disable_semaphore_checks=True,
        ),
    )(xt, so)
    return (out, sentences_per_doc)


# speedup vs baseline: 1.0123x; 1.0060x over previous
import jax
import jax.numpy as jnp
from jax import lax
from jax.experimental import pallas as pl
from jax.experimental.pallas import tpu as pltpu
from jax.experimental.pallas import tpu_sc as plsc

B = 4096
S = 50
D = 768
NC = 2
NS = 16
NW = NC * NS
B_PER_W = B // NW


CH = 4
CB = B_PER_W // CH


def _body(xt_ref, so_ref, out_ref, dst_idx, rows, sem_g, sem_i, sem_s):
    wid = lax.axis_index("c") * NS + lax.axis_index("s")
    base = wid * B_PER_W
    gathers = [
        pltpu.async_copy(
            xt_ref.at[0, pl.ds(base + k * CB, CB)],
            rows.at[pl.ds(k * CB, CB)],
            sem_g.at[k],
        )
        for k in range(CH)
    ]
    idx_copies = [
        pltpu.async_copy(so_ref.at[pl.ds(base + k * CB, CB)], dst_idx.at[k], sem_i)
        for k in range(CH)
    ]
    for c in idx_copies:
        c.wait()
    scatters = []
    for k in range(CH):
        gathers[k].wait()
        scatters.append(
            pltpu.async_copy(rows.at[pl.ds(k * CB, CB)], out_ref.at[dst_idx.at[k]], sem_s)
        )
    for c in scatters:
        c.wait()


def kernel(embeded_big_tensor, sorted_lengths, sort_order, sentences_per_doc):
    xt = embeded_big_tensor.transpose(1, 0, 2)
    so = sort_order.astype(jnp.int32)
    mesh = plsc.VectorSubcoreMesh(core_axis_name="c", subcore_axis_name="s")
    out = pl.kernel(
        _body,
        out_type=jax.ShapeDtypeStruct((B, D), jnp.float32),
        mesh=mesh,
        scratch_types=[
            pltpu.VMEM((CH, CB), jnp.int32),
            pltpu.VMEM((B_PER_W, D), jnp.float32),
            pltpu.SemaphoreType.DMA((CH,)),
            pltpu.SemaphoreType.DMA,
            pltpu.SemaphoreType.DMA,
        ],
        compiler_params=pltpu.CompilerParams(
            skip_device_barrier=True,
            disable_bounds_checks=True,
            disable_semaphore_checks=True,
        ),
    )(xt, so)
    return (out, sentences_per_doc)
